# baseline (speedup 1.0000x reference)
import functools

import jax
import jax.numpy as jnp
from jax import lax
from jax.experimental import pallas as pl
from jax.experimental.pallas import tpu as pltpu
from jax.experimental.pallas import tpu_sc as plsc

_NUM_CORES = 2
_NUM_SUBCORES = 16
_NUM_WORKERS = _NUM_CORES * _NUM_SUBCORES
_LANES = 16


def _make_gather(embed, batch):
    b_per_w = batch // _NUM_WORKERS
    mesh = plsc.VectorSubcoreMesh(core_axis_name="c", subcore_axis_name="s")

    @functools.partial(
        pl.kernel,
        mesh=mesh,
        out_type=jax.ShapeDtypeStruct((batch, embed), jnp.float32),
        scratch_types=[
            pltpu.VMEM((b_per_w,), jnp.int32),
            pltpu.VMEM((b_per_w, embed), jnp.float32),
            pltpu.SemaphoreType.DMA,
            pltpu.SemaphoreType.DMA,
            pltpu.SemaphoreType.DMA,
            pltpu.SemaphoreType.DMA,
        ],
    )
    def gather_kernel(table_hbm, idx_hbm, out_hbm, ids_v, out_v,
                      sem0, sem1, sem2, sem3):
        sems = (sem0, sem1, sem2, sem3)
        wid = lax.axis_index("s") * _NUM_CORES + lax.axis_index("c")
        base = wid * b_per_w
        pltpu.sync_copy(idx_hbm.at[pl.ds(base, b_per_w)], ids_v)

        half = b_per_w // 2

        @plsc.parallel_loop(0, half // _LANES, 1, unroll=2)
        def row_body_a(g):
            ids16 = ids_v[pl.ds(g * _LANES, _LANES)]
            for k in range(_LANES):
                pltpu.async_copy(
                    table_hbm.at[ids16[k]],
                    out_v.at[g * _LANES + k],
                    sems[0],
                )

        @plsc.parallel_loop(half // _LANES, b_per_w // _LANES, 1, unroll=2)
        def row_body_b(g):
            ids16 = ids_v[pl.ds(g * _LANES, _LANES)]
            for k in range(_LANES):
                pltpu.async_copy(
                    table_hbm.at[ids16[k]],
                    out_hbm.at[base + g * _LANES + k],
                    sems[1],
                )

        pltpu.make_async_copy(
            table_hbm.at[pl.ds(0, half)], out_v.at[pl.ds(0, half)], sems[0]
        ).wait()
        pltpu.sync_copy(
            out_v.at[pl.ds(0, half)], out_hbm.at[pl.ds(base, half)]
        )
        pltpu.make_async_copy(
            table_hbm.at[pl.ds(0, half)],
            out_hbm.at[pl.ds(base + half, half)],
            sems[1],
        ).wait()

    return gather_kernel


def kernel(node_embs, node_ids):
    _, embed = node_embs.shape
    (batch,) = node_ids.shape
    gather = _make_gather(embed, batch)
    return gather(node_embs, node_ids.astype(jnp.int32))


# speedup vs baseline: 1.2792x; 1.2792x over previous
import functools

import jax
import jax.numpy as jnp
from jax import lax
from jax.experimental import pallas as pl
from jax.experimental.pallas import tpu as pltpu
from jax.experimental.pallas import tpu_sc as plsc

_NUM_CORES = 2
_NUM_SUBCORES = 16
_NUM_WORKERS = _NUM_CORES * _NUM_SUBCORES
_LANES = 16


def _make_gather(embed, batch):
    b_per_w = batch // _NUM_WORKERS
    half = b_per_w // 2
    mesh = plsc.VectorSubcoreMesh(core_axis_name="c", subcore_axis_name="s")

    @functools.partial(
        pl.kernel,
        mesh=mesh,
        out_type=jax.ShapeDtypeStruct((batch, embed), jnp.float32),
        scratch_types=[
            pltpu.VMEM((b_per_w,), jnp.int32),
            pltpu.VMEM((half, embed), jnp.float32),
            pltpu.VMEM_SHARED((_NUM_SUBCORES, half, embed), jnp.float32),
            pltpu.SemaphoreType.DMA,
            pltpu.SemaphoreType.DMA,
        ],
    )
    def gather_kernel(table_hbm, idx_hbm, out_hbm, ids_v, out_v,
                      shared_v, sem0, sem1):
        sid = lax.axis_index("s")
        wid = sid * _NUM_CORES + lax.axis_index("c")
        base = wid * b_per_w
        pltpu.sync_copy(idx_hbm.at[pl.ds(base, b_per_w)], ids_v)

        @plsc.parallel_loop(0, half // _LANES, 1, unroll=2)
        def row_body_a(g):
            ids16 = ids_v[pl.ds(g * _LANES, _LANES)]
            for k in range(_LANES):
                pltpu.async_copy(
                    table_hbm.at[ids16[k]],
                    out_v.at[g * _LANES + k],
                    sem0,
                )

        @plsc.parallel_loop(half // _LANES, b_per_w // _LANES, 1, unroll=2)
        def row_body_b(g):
            ids16 = ids_v[pl.ds(g * _LANES, _LANES)]
            for k in range(_LANES):
                pltpu.async_copy(
                    table_hbm.at[ids16[k]],
                    shared_v.at[sid, g * _LANES + k - half],
                    sem1,
                )

        pltpu.make_async_copy(
            table_hbm.at[pl.ds(0, half)], out_v, sem0
        ).wait()
        pltpu.sync_copy(out_v, out_hbm.at[pl.ds(base, half)])
        pltpu.make_async_copy(
            table_hbm.at[pl.ds(0, half)], shared_v.at[sid], sem1
        ).wait()
        pltpu.sync_copy(
            shared_v.at[sid], out_hbm.at[pl.ds(base + half, half)]
        )

    return gather_kernel


def kernel(node_embs, node_ids):
    _, embed = node_embs.shape
    (batch,) = node_ids.shape
    gather = _make_gather(embed, batch)
    return gather(node_embs, node_ids.astype(jnp.int32))


# speedup vs baseline: 1.3305x; 1.0400x over previous
import functools

import jax
import jax.numpy as jnp
from jax import lax
from jax.experimental import pallas as pl
from jax.experimental.pallas import tpu as pltpu
from jax.experimental.pallas import tpu_sc as plsc

_NUM_CORES = 2
_NUM_SUBCORES = 16
_NUM_WORKERS = _NUM_CORES * _NUM_SUBCORES
_LANES = 16


def _make_gather(embed, batch):
    b_per_w = batch // _NUM_WORKERS
    mesh = plsc.VectorSubcoreMesh(core_axis_name="c", subcore_axis_name="s")

    @functools.partial(
        pl.kernel,
        mesh=mesh,
        out_type=jax.ShapeDtypeStruct((batch, embed), jnp.float32),
        scratch_types=[
            pltpu.VMEM((b_per_w,), jnp.int32),
            pltpu.VMEM((b_per_w, embed), jnp.float32),
            pltpu.SemaphoreType.DMA,
            pltpu.SemaphoreType.DMA,
            pltpu.SemaphoreType.DMA,
            pltpu.SemaphoreType.DMA,
        ],
    )
    def gather_kernel(table_hbm, idx_hbm, out_hbm, ids_v, out_v,
                      sem0, sem1, sem2, sem3):
        sems = (sem0, sem1, sem2, sem3)
        wid = lax.axis_index("s") * _NUM_CORES + lax.axis_index("c")
        base = wid * b_per_w
        pltpu.sync_copy(idx_hbm.at[pl.ds(base, b_per_w)], ids_v)

        @plsc.parallel_loop(0, b_per_w // _LANES, 1, unroll=2)
        def row_body(g):
            ids16 = ids_v[pl.ds(g * _LANES, _LANES)]
            for k in range(_LANES):
                pltpu.async_copy(
                    table_hbm.at[ids16[k]],
                    out_v.at[g * _LANES + k],
                    sems[k % 4],
                )
        for s in sems:
            pltpu.make_async_copy(
                table_hbm.at[pl.ds(0, b_per_w // 4)],
                out_v.at[pl.ds(0, b_per_w // 4)],
                s,
            ).wait()
        pltpu.sync_copy(out_v, out_hbm.at[pl.ds(base, b_per_w)])

    return gather_kernel


def kernel(node_embs, node_ids):
    _, embed = node_embs.shape
    (batch,) = node_ids.shape
    gather = _make_gather(embed, batch)
    return gather(node_embs, node_ids.astype(jnp.int32))
